# baseline (speedup 1.0000x reference)
import functools

import jax
import jax.numpy as jnp
import numpy as np
from jax import lax
from jax.experimental import pallas as pl
from jax.experimental.pallas import tpu as pltpu
from jax.experimental.pallas import tpu_sc as plsc

_N_BINS = 15
_N_ROWS = 65536
_N_COLS = 1000
_NC = 2
_NS = 16
_NW = _NC * _NS
_ROWS_PER_W = _N_ROWS // _NW
_B = 16
_SLABS = _ROWS_PER_W // _B
_NBUF = 4
_PITCH = 1009
_CH = 4


def _sc_body(logits_hbm, labels_hbm, out_hbm, slab_v, lab_v, acc_v, flat_v,
             sems):
    wid = lax.axis_index("s") * _NC + lax.axis_index("c")
    base = wid * _ROWS_PER_W

    for q in range(3 * (_N_BINS + 1)):
        acc_v[q] = jnp.zeros((16,), jnp.float32)

    lane = lax.broadcasted_iota(jnp.int32, (16,), 0)
    step_f = np.float32(1.0) / np.float32(_N_BINS)
    zero = jnp.zeros((16,), jnp.float32)
    one = jnp.ones((16,), jnp.float32)
    neg_inf = jnp.full((16,), -jnp.inf, jnp.float32)
    lane_pitch = lane * _PITCH

    def _copy(t, b):
        row0 = base + t * _B
        log_cp = pltpu.make_async_copy(
            logits_hbm.at[pl.ds(row0, _B), :],
            slab_v.at[b],
            sems.at[b, 0])
        lab_cp = pltpu.make_async_copy(
            labels_hbm.at[pl.ds(row0, _B)],
            lab_v.at[b],
            sems.at[b, 1])
        return log_cp, lab_cp

    for t0 in range(_NBUF - 1):
        ca, cb = _copy(t0, t0)
        ca.start()
        cb.start()

    _CSTARTS = [16 * c for c in range(62)] + [984]

    def slab_loop(t, carry):
        b = lax.rem(t, _NBUF)

        @pl.when(t + (_NBUF - 1) < _SLABS)
        def _prefetch():
            ca, cb = _copy(t + (_NBUF - 1), lax.rem(t + (_NBUF - 1), _NBUF))
            ca.start()
            cb.start()

        ca, cb = _copy(t, b)
        ca.wait()
        cb.wait()

        @plsc.parallel_loop(0, _B, 1, unroll=4)
        def _repitch(r):
            rp = r * _PITCH
            for cs in _CSTARTS:
                flat_v[pl.ds(rp + cs, 16)] = slab_v[b, r, pl.ds(cs, 16)]

        init = (tuple(zero for _ in range(_CH)),
                tuple(neg_inf for _ in range(_CH)),
                tuple(lane_pitch + u for u in range(_CH)))

        @plsc.parallel_loop(0, _N_COLS // _CH, 1, unroll=4, carry=init)
        def col_result(j, c):
            ss = list(c[0])
            mm = list(c[1])
            aa = list(c[2])
            for u in range(_CH):
                v = plsc.load_gather(flat_v, [aa[u]])
                ss[u] = ss[u] + jnp.exp(v)
                mm[u] = jnp.maximum(mm[u], v)
                aa[u] = aa[u] + _CH
            return (tuple(ss), tuple(mm), tuple(aa))

        ss, mm, _ = col_result
        ss, mm = list(ss), list(mm)
        s_vec = (ss[0] + ss[1]) + (ss[2] + ss[3])
        m_vec = jnp.maximum(jnp.maximum(mm[0], mm[1]),
                            jnp.maximum(mm[2], mm[3]))

        conf = jnp.exp(m_vec) / s_vec
        labs = lab_v[b]
        l_lab = plsc.load_gather(flat_v, [lane_pitch + labs])
        accf = jnp.where(l_lab == m_vec, one, zero)

        for i in range(_N_BINS):
            lo = np.float32(i) * step_f
            hi = np.float32(i + 1) * step_f
            in_i = jnp.logical_and(conf > lo, conf <= hi)
            plsc.addupdate(acc_v.at[i], jnp.where(in_i, one, zero))
            plsc.addupdate(acc_v.at[16 + i], jnp.where(in_i, conf, zero))
            plsc.addupdate(acc_v.at[32 + i], jnp.where(in_i, accf, zero))
        return carry

    lax.fori_loop(0, _SLABS, slab_loop, 0)
    pltpu.sync_copy(acc_v, out_hbm.at[wid])


def kernel(logits, labels):
    labels_i = labels.astype(jnp.int32)

    mesh = plsc.VectorSubcoreMesh(core_axis_name="c", subcore_axis_name="s")
    partials = pl.kernel(
        _sc_body,
        out_type=jax.ShapeDtypeStruct((_NW, 3 * (_N_BINS + 1), 16), jnp.float32),
        mesh=mesh,
        scratch_types=[
            pltpu.VMEM((_NBUF, _B, _N_COLS), jnp.float32),
            pltpu.VMEM((_NBUF, _B), jnp.int32),
            pltpu.VMEM((3 * (_N_BINS + 1), 16), jnp.float32),
            pltpu.VMEM((_B * _PITCH + 16,), jnp.float32),
            pltpu.SemaphoreType.DMA((_NBUF, 2)),
        ],
        compiler_params=pltpu.CompilerParams(needs_layout_passes=False),
    )(logits, labels_i)

    s = jnp.sum(partials, axis=(0, 2))
    cnt = s[0:_N_BINS]
    conf_s = s[16:16 + _N_BINS]
    acc_s = s[32:32 + _N_BINS]
    cnt_safe = jnp.maximum(cnt, 1.0)
    prop = cnt / _N_ROWS
    contrib = jnp.abs(conf_s / cnt_safe - acc_s / cnt_safe) * prop
    ece = jnp.sum(jnp.where(prop > 0, contrib, 0.0))
    return ece.reshape(1)


# speedup vs baseline: 1.2982x; 1.2982x over previous
import functools

import jax
import jax.numpy as jnp
import numpy as np
from jax import lax
from jax.experimental import pallas as pl
from jax.experimental.pallas import tpu as pltpu
from jax.experimental.pallas import tpu_sc as plsc

_N_BINS = 15
_N_ROWS = 65536
_N_COLS = 1000

_SC_ROWS = 28672
_TC_ROWS = _N_ROWS - _SC_ROWS

_NC = 2
_NS = 16
_NW = _NC * _NS
_ROWS_PER_W = _SC_ROWS // _NW
_B = 32
_SLABS = _ROWS_PER_W // _B
_GROUPS = _B // 16
_PITCH = 1009
_CH = 4

_TR = 512
_TW = 4
_TC_GRID = _TC_ROWS // (_TR * _TW)
_TC_BLK0 = _SC_ROWS // _TR


def _sc_body(logits_hbm, labels_hbm, out_hbm, slab_v, lab_v, acc_v, flat_v,
             sems):
    wid = lax.axis_index("s") * _NC + lax.axis_index("c")
    base = wid * _ROWS_PER_W

    for q in range(3 * (_N_BINS + 1)):
        acc_v[q] = jnp.zeros((16,), jnp.float32)

    lane = lax.broadcasted_iota(jnp.int32, (16,), 0)
    step_f = np.float32(1.0) / np.float32(_N_BINS)
    zero = jnp.zeros((16,), jnp.float32)
    one = jnp.ones((16,), jnp.float32)
    neg_inf = jnp.full((16,), -jnp.inf, jnp.float32)
    lane_pitch = lane * _PITCH

    def _copy(t, b):
        row0 = base + t * _B
        log_cp = pltpu.make_async_copy(
            logits_hbm.at[pl.ds(row0, _B), :],
            slab_v.at[b],
            sems.at[b, 0])
        lab_cp = pltpu.make_async_copy(
            labels_hbm.at[pl.ds(row0, _B)],
            lab_v.at[b],
            sems.at[b, 1])
        return log_cp, lab_cp

    c0a, c0b = _copy(0, 0)
    c0a.start()
    c0b.start()

    _CSTARTS = [16 * c for c in range(62)] + [984]

    def slab_loop(t, carry):
        b = lax.rem(t, 2)

        @pl.when(t + 1 < _SLABS)
        def _prefetch():
            ca, cb = _copy(t + 1, 1 - b)
            ca.start()
            cb.start()

        ca, cb = _copy(t, b)
        ca.wait()
        cb.wait()

        @plsc.parallel_loop(0, _B, 1, unroll=4)
        def _repitch(r):
            rp = r * _PITCH
            for cs in _CSTARTS:
                flat_v[pl.ds(rp + cs, 16)] = slab_v[b, r, pl.ds(cs, 16)]

        for g in range(_GROUPS):
            gbase = g * 16 * _PITCH
            init = (tuple(zero for _ in range(_CH)),
                    tuple(neg_inf for _ in range(_CH)),
                    tuple(lane_pitch + (gbase + u) for u in range(_CH)))

            @plsc.parallel_loop(0, _N_COLS // _CH, 1, unroll=4, carry=init)
            def col_result(j, c):
                ss = list(c[0])
                mm = list(c[1])
                aa = list(c[2])
                for u in range(_CH):
                    v = plsc.load_gather(flat_v, [aa[u]])
                    ss[u] = ss[u] + jnp.exp(v)
                    mm[u] = jnp.maximum(mm[u], v)
                    aa[u] = aa[u] + _CH
                return (tuple(ss), tuple(mm), tuple(aa))

            ss, mm, _ = col_result
            ss, mm = list(ss), list(mm)
            s_vec = (ss[0] + ss[1]) + (ss[2] + ss[3])
            m_vec = jnp.maximum(jnp.maximum(mm[0], mm[1]),
                                jnp.maximum(mm[2], mm[3]))

            conf = jnp.exp(m_vec) / s_vec
            labs = lab_v[b, pl.ds(g * 16, 16)]
            l_lab = plsc.load_gather(flat_v, [lane_pitch + (labs + gbase)])
            accf = jnp.where(l_lab == m_vec, one, zero)

            for i in range(_N_BINS):
                lo = np.float32(i) * step_f
                hi = np.float32(i + 1) * step_f
                in_i = jnp.logical_and(conf > lo, conf <= hi)
                plsc.addupdate(acc_v.at[i], jnp.where(in_i, one, zero))
                plsc.addupdate(acc_v.at[16 + i], jnp.where(in_i, conf, zero))
                plsc.addupdate(acc_v.at[32 + i], jnp.where(in_i, accf, zero))
        return carry

    lax.fori_loop(0, _SLABS, slab_loop, 0)
    pltpu.sync_copy(acc_v, out_hbm.at[wid])


def _tc_slab_stats(l, lab):
    m = jnp.max(l, axis=1, keepdims=True)
    e = jnp.exp(l - m)
    s = jnp.sum(e, axis=1, keepdims=True)
    conf = 1.0 / s
    idx = lax.broadcasted_iota(jnp.int32, l.shape, 1)
    l_at_lab = jnp.max(jnp.where(idx == lab, l, -jnp.inf), axis=1,
                       keepdims=True)
    accf = (l_at_lab == m).astype(jnp.float32)
    return conf, accf


def _tc_bin_partials(conf, accf):
    step_f = jnp.float32(1.0) / jnp.float32(_N_BINS)
    k = lax.broadcasted_iota(jnp.int32, (1, _N_BINS + 1), 1).astype(jnp.float32)
    lo = k * step_f
    hi = jnp.where(k >= _N_BINS, jnp.float32(jnp.inf), (k + 1.0) * step_f)
    onehot = jnp.logical_and(conf > lo, conf <= hi).astype(jnp.float32)
    cnt_p = jnp.sum(onehot, axis=0, keepdims=True)
    conf_p = jnp.sum(onehot * conf, axis=0, keepdims=True)
    acc_p = jnp.sum(onehot * accf, axis=0, keepdims=True)
    return cnt_p, conf_p, acc_p


def _tc_body(*refs):
    logits_refs = refs[:_TW]
    labels_refs = refs[_TW:2 * _TW]
    out_ref = refs[2 * _TW]
    acc_ref = refs[2 * _TW + 1]
    step = pl.program_id(0)

    @pl.when(step == 0)
    def _init():
        acc_ref[...] = jnp.zeros_like(acc_ref)

    cnt_t = jnp.zeros((1, _N_BINS + 1), jnp.float32)
    conf_t = jnp.zeros((1, _N_BINS + 1), jnp.float32)
    acc_t = jnp.zeros((1, _N_BINS + 1), jnp.float32)
    for w in range(_TW):
        conf, accf = _tc_slab_stats(logits_refs[w][...], labels_refs[w][0])
        cnt_p, conf_p, acc_p = _tc_bin_partials(conf, accf)
        cnt_t += cnt_p
        conf_t += conf_p
        acc_t += acc_p
    acc_ref[0:1, :] += cnt_t
    acc_ref[1:2, :] += conf_t
    acc_ref[2:3, :] += acc_t

    @pl.when(step == _TC_GRID - 1)
    def _final():
        out_ref[...] = acc_ref[...]


def kernel(logits, labels):
    labels_i = labels.astype(jnp.int32)

    mesh = plsc.VectorSubcoreMesh(core_axis_name="c", subcore_axis_name="s")
    sc_partials = pl.kernel(
        _sc_body,
        out_type=jax.ShapeDtypeStruct((_NW, 3 * (_N_BINS + 1), 16),
                                      jnp.float32),
        mesh=mesh,
        scratch_types=[
            pltpu.VMEM((2, _B, _N_COLS), jnp.float32),
            pltpu.VMEM((2, _B), jnp.int32),
            pltpu.VMEM((3 * (_N_BINS + 1), 16), jnp.float32),
            pltpu.VMEM((_B * _PITCH + 16,), jnp.float32),
            pltpu.SemaphoreType.DMA((2, 2)),
        ],
        compiler_params=pltpu.CompilerParams(needs_layout_passes=False),
    )(logits, labels_i)

    labels3 = labels_i.reshape(_N_ROWS // _TR, _TR, 1)
    in_specs = (
        [pl.BlockSpec(
            (_TR, _N_COLS),
            functools.partial(lambda i, w: (_TC_BLK0 + _TW * i + w, 0), w=w))
         for w in range(_TW)]
        + [pl.BlockSpec(
            (1, _TR, 1),
            functools.partial(lambda i, w: (_TC_BLK0 + _TW * i + w, 0, 0), w=w))
           for w in range(_TW)]
    )
    tc_partials = pl.pallas_call(
        _tc_body,
        grid=(_TC_GRID,),
        in_specs=in_specs,
        out_specs=pl.BlockSpec((8, _N_BINS + 1), lambda i: (0, 0)),
        out_shape=jax.ShapeDtypeStruct((8, _N_BINS + 1), jnp.float32),
        scratch_shapes=[pltpu.VMEM((8, _N_BINS + 1), jnp.float32)],
    )(*([logits] * _TW), *([labels3] * _TW))

    s_sc = jnp.sum(sc_partials, axis=(0, 2))
    cnt = s_sc[0:_N_BINS] + tc_partials[0, :_N_BINS]
    conf_s = s_sc[16:16 + _N_BINS] + tc_partials[1, :_N_BINS]
    acc_s = s_sc[32:32 + _N_BINS] + tc_partials[2, :_N_BINS]
    cnt_safe = jnp.maximum(cnt, 1.0)
    prop = cnt / _N_ROWS
    contrib = jnp.abs(conf_s / cnt_safe - acc_s / cnt_safe) * prop
    ece = jnp.sum(jnp.where(prop > 0, contrib, 0.0))
    return ece.reshape(1)
